# baseline (speedup 1.0000x reference)
import functools

import jax
import jax.numpy as jnp
from jax import lax
from jax.experimental import pallas as pl
from jax.experimental.pallas import tpu as pltpu
from jax.experimental.pallas import tpu_sc as plsc

B = 4096
L = 20
K = 64
NC = 2
NS = 16
NW = NC * NS
BPW = B // NW
G = (BPW * L) // 128
PADR = 8
REG = BPW + PADR
NPAIR = 50000
GS = 24
RS = 16

_mesh = plsc.VectorSubcoreMesh(core_axis_name="c", subcore_axis_name="s")


@functools.partial(
    pl.kernel,
    mesh=_mesh,
    out_type=(jax.ShapeDtypeStruct((B, 2 * K), jnp.float32),
              jax.ShapeDtypeStruct((B, 2 * K), jnp.float32)),
    scratch_types=[
        pltpu.VMEM((GS, 128), jnp.int32),
        pltpu.VMEM((GS, 128), jnp.int32),
        pltpu.VMEM((GS, 128), jnp.int32),
        pltpu.VMEM((128, 2 * K), jnp.float32),
        pltpu.VMEM((128, 2 * K), jnp.float32),
        pltpu.VMEM_SHARED((NS * REG, 2 * K), jnp.float32),
        pltpu.VMEM_SHARED((NS * REG, 2 * K), jnp.float32),
        pltpu.SemaphoreType.DMA,
        pltpu.SemaphoreType.DMA,
    ],
)
def _sc_pool(LIp, mh, sah, sbh, zrh, outa, outb,
             bkv, pav, pbv, gbufa, gbufb, sha, shb, sga, sgb):
    wid = lax.axis_index("s") * NC + lax.axis_index("c")
    sid = lax.axis_index("s")
    base = wid * BPW
    bufs = (gbufa, gbufb)
    sems = (sga, sgb)

    pltpu.sync_copy(mh.at[pl.ds(wid * GS, GS)], bkv)
    pltpu.sync_copy(sah.at[pl.ds(wid * GS, GS)], pav)
    pltpu.sync_copy(sbh.at[pl.ds(wid * GS, GS)], pbv)
    pltpu.sync_copy(zrh, sha.at[pl.ds(sid * REG, REG)])
    pltpu.sync_copy(zrh, shb.at[pl.ds(sid * REG, REG)])

    pend = [None, None]
    pend[0] = pltpu.async_copy(LIp.at[bkv.at[0]], bufs[0], sems[0])
    for g in range(G):
        if g + 1 < G:
            pend[(g + 1) % 2] = pltpu.async_copy(
                LIp.at[bkv.at[g + 1]], bufs[(g + 1) % 2], sems[(g + 1) % 2])
        pend[g % 2].wait()
        pltpu.sync_copy(bufs[g % 2], sha.at[pav.at[g]], add=True)
        pltpu.sync_copy(bufs[g % 2], shb.at[pbv.at[g]], add=True)

    pltpu.sync_copy(sha.at[pl.ds(sid * REG, BPW)], outa.at[pl.ds(base, BPW)])
    pltpu.sync_copy(shb.at[pl.ds(sid * REG, BPW)], outb.at[pl.ds(base, BPW)])


@functools.partial(
    pl.kernel,
    mesh=_mesh,
    out_type=jax.ShapeDtypeStruct((20, B, 2 * K), jnp.float32),
    scratch_types=[
        pltpu.VMEM((RS, 128), jnp.int32),
        pltpu.VMEM((128, 2 * K), jnp.float32),
        pltpu.VMEM((128, 2 * K), jnp.float32),
        pltpu.SemaphoreType.DMA,
        pltpu.SemaphoreType.DMA,
    ],
)
def _sc_rows(IL1, IL2, IL3, IL4, UI1, UI2, UI3, UI4, IU, idxh,
             out, idxv, gbufa, gbufb, sga, sgb):
    wid = lax.axis_index("s") * NC + lax.axis_index("c")
    base = wid * BPW
    bufs = (gbufa, gbufb)
    gsems = (sga, sgb)

    pltpu.sync_copy(idxh.at[pl.ds(wid * RS, RS)], idxv)

    ILs = (IL1, IL2, IL3, IL4)
    UIs = (UI1, UI2, UI3, UI4)
    plan = []
    for i in range(4):
        plan.append((ILs[i], 1 + i, i))
        plan.append((ILs[i], 5 + i, 4 + i))
        plan.append((UIs[i], 0, 8 + i))
        plan.append((IU, 1 + i, 12 + i))
        plan.append((IU, 5 + i, 16 + i))

    n = len(plan)
    gp = [None, None]
    gp[0] = pltpu.async_copy(plan[0][0].at[idxv.at[plan[0][1]]], bufs[0],
                             gsems[0])
    for j in range(n):
        if j + 1 < n:
            tbl, r, _ = plan[j + 1]
            gp[(j + 1) % 2] = pltpu.async_copy(
                tbl.at[idxv.at[r]], bufs[(j + 1) % 2], gsems[(j + 1) % 2])
        gp[j % 2].wait()
        slot = plan[j][2]
        pltpu.sync_copy(bufs[j % 2], out.at[slot, pl.ds(base, BPW)])


_BB = 512

_HROW = ([1 + i for i in range(4)] + [5 + i for i in range(4)]
         + [0] * 4 + [1 + i for i in range(4)] + [5 + i for i in range(4)])


def _tc_body(fa1, fa2, fa3, fa4, fb1, fb2, fb3, fb4, r_ref, h_ref, al_ref,
             o_ref):
    fmc = []
    for fa, fb in zip((fa1, fa2, fa3, fa4), (fb1, fb2, fb3, fb4)):
        a = fa[...]
        b = fb[...]
        fmc.append((a[:, 0:K] + b[:, K:2 * K]) * jnp.float32(1.0 / L))
    r = r_ref[...]
    h = h_ref[...]
    a4 = jax.nn.sigmoid(al_ref[0, :])
    neg_inf = jnp.float32(-2.0 ** 32 + 1)

    def pick(slot):
        full = r[slot]
        hs = h[_HROW[slot]][:, None]
        return jnp.where(hs == 0, full[:, 0:K], full[:, K:2 * K])

    for i in range(4):
        u = pick(8 + i)
        for sgn in range(2):
            e = pick(4 * sgn + i)
            iu = pick(12 + 4 * sgn + i)
            d = jnp.concatenate(
                [jnp.sum(fmc[t] * e, axis=1, keepdims=True) for t in range(4)],
                axis=1)
            w = d * jnp.float32(0.125)
            w = jnp.where(w == 0.0, neg_inf, w)
            w = w - jnp.max(w, axis=1, keepdims=True)
            p = jnp.exp(w)
            p = p / jnp.sum(p, axis=1, keepdims=True)
            att = jnp.sum(p * d, axis=1)
            mf = jnp.sum(u * iu, axis=1)
            o_ref[2 * i + sgn, :] = a4[i] * att + (1.0 - a4[i]) * mf


def _depad(t):
    return t.reshape(-1)[:NPAIR * 2 * K].reshape(NPAIR, 2 * K)


def _stage(rows, stride):
    n = rows.shape[1]
    pad = jnp.zeros((NW, stride - n, 128), jnp.int32)
    return jnp.concatenate([rows, pad], axis=1).reshape(NW * stride, 128)


def kernel(uid, basket_1, basket_2, basket_3, basket_4,
           iid_1, iid_2, iid_3, iid_4,
           neg_iid_1, neg_iid_2, neg_iid_3, neg_iid_4,
           IL_1, IL_2, IL_3, IL_4, LI_1, LI_2, LI_3, LI_4,
           UI_1, UI_2, UI_3, UI_4, IU,
           alpha_mor, alpha_aft, alpha_eve, alpha_deep):
    i32 = jnp.int32

    f = jnp.arange(B * L, dtype=i32)
    wvec = f // (BPW * L)
    srel = (f % (BPW * L)) // L
    sidv = wvec // NC
    slot = sidv * REG + srel
    trash = sidv * REG + BPW + (f % PADR)

    mhs, sahs, sbhs = [], [], []
    for bk in (basket_1, basket_2, basket_3, basket_4):
        bkf = bk.reshape(-1).astype(i32)
        m = bkf // 2
        h = bkf & 1
        mhs.append(_stage(m.reshape(NW, G, 128), GS))
        sahs.append(_stage(jnp.where(h == 0, slot, trash).reshape(NW, G, 128),
                           GS))
        sbhs.append(_stage(jnp.where(h == 1, slot, trash).reshape(NW, G, 128),
                           GS))
    zrh = jnp.zeros((REG, 2 * K), jnp.float32)

    idx9 = jnp.stack([uid, iid_1, iid_2, iid_3, iid_4,
                      neg_iid_1, neg_iid_2, neg_iid_3, neg_iid_4]).astype(i32)
    idxp = (idx9 // 2).reshape(9, NW, 128).transpose(1, 0, 2)
    idxh = _stage(idxp, RS)
    h9 = idx9 & 1

    LIp = [_depad(t) for t in (LI_1, LI_2, LI_3, LI_4)]
    ILp = [_depad(t) for t in (IL_1, IL_2, IL_3, IL_4)]
    UIp = [_depad(t) for t in (UI_1, UI_2, UI_3, UI_4)]
    IUp = _depad(IU)

    pooled = [_sc_pool(LIp[i], mhs[i], sahs[i], sbhs[i], zrh)
              for i in range(4)]
    rows = _sc_rows(*ILp, *UIp, IUp, idxh)

    alphas = jnp.stack([alpha_mor, alpha_aft, alpha_eve, alpha_deep])
    alphas = alphas.astype(jnp.float32).reshape(1, 4)

    fspec = pl.BlockSpec((_BB, 2 * K), lambda j: (j, 0))
    out = pl.pallas_call(
        _tc_body,
        grid=(B // _BB,),
        in_specs=[fspec] * 8 + [
            pl.BlockSpec((20, _BB, 2 * K), lambda j: (0, j, 0)),
            pl.BlockSpec((9, _BB), lambda j: (0, j)),
            pl.BlockSpec((1, 4), lambda j: (0, 0))],
        out_specs=pl.BlockSpec((8, _BB), lambda j: (0, j)),
        out_shape=jax.ShapeDtypeStruct((8, B), jnp.float32),
    )(*[p[0] for p in pooled], *[p[1] for p in pooled], rows, h9, alphas)

    return tuple(out[i] for i in range(8))


# speedup vs baseline: 1.0067x; 1.0067x over previous
import functools

import jax
import jax.numpy as jnp
from jax import lax
from jax.experimental import pallas as pl
from jax.experimental.pallas import tpu as pltpu
from jax.experimental.pallas import tpu_sc as plsc

B = 4096
L = 20
K = 64
NC = 2
NS = 16
NW = NC * NS
BPW = B // NW
G = (BPW * L) // 128
VPAD = 100008
GS = 24
RS = 16

_mesh = plsc.VectorSubcoreMesh(core_axis_name="c", subcore_axis_name="s")


@functools.partial(
    pl.kernel,
    mesh=_mesh,
    out_type=jax.ShapeDtypeStruct((B, 2 * K), jnp.float32),
    scratch_types=[
        pltpu.VMEM((GS, 128), jnp.int32),
        pltpu.VMEM((GS, 128), jnp.int32),
        pltpu.VMEM((128, 2 * K), jnp.float32),
        pltpu.VMEM((128, 2 * K), jnp.float32),
        pltpu.VMEM_SHARED((NS * BPW, 2 * K), jnp.float32),
        pltpu.SemaphoreType.DMA,
        pltpu.SemaphoreType.DMA,
    ],
)
def _sc_pool(LIp, mh, sah, zrh, out, bkv, pav, gbufa, gbufb, sha, sga, sgb):
    wid = lax.axis_index("s") * NC + lax.axis_index("c")
    sid = lax.axis_index("s")
    base = wid * BPW
    bufs = (gbufa, gbufb)
    sems = (sga, sgb)

    pltpu.sync_copy(mh.at[pl.ds(wid * GS, GS)], bkv)
    pltpu.sync_copy(sah.at[pl.ds(wid * GS, GS)], pav)
    pltpu.sync_copy(zrh, sha.at[pl.ds(sid * BPW, BPW)])

    pend = [None, None]
    pend[0] = pltpu.async_copy(LIp.at[bkv.at[0]], bufs[0], sems[0])
    for g in range(G):
        if g + 1 < G:
            pend[(g + 1) % 2] = pltpu.async_copy(
                LIp.at[bkv.at[g + 1]], bufs[(g + 1) % 2], sems[(g + 1) % 2])
        pend[g % 2].wait()
        pltpu.sync_copy(bufs[g % 2], sha.at[pav.at[g]], add=True)

    pltpu.sync_copy(sha.at[pl.ds(sid * BPW, BPW)], out.at[pl.ds(base, BPW)])


@functools.partial(
    pl.kernel,
    mesh=_mesh,
    out_type=jax.ShapeDtypeStruct((20, B, 2 * K), jnp.float32),
    scratch_types=[
        pltpu.VMEM((RS, 128), jnp.int32),
        pltpu.VMEM((128, 2 * K), jnp.float32),
        pltpu.VMEM((128, 2 * K), jnp.float32),
        pltpu.SemaphoreType.DMA,
        pltpu.SemaphoreType.DMA,
    ],
)
def _sc_rows(IL1, IL2, IL3, IL4, UI1, UI2, UI3, UI4, IU, idxh,
             out, idxv, gbufa, gbufb, sga, sgb):
    wid = lax.axis_index("s") * NC + lax.axis_index("c")
    base = wid * BPW
    bufs = (gbufa, gbufb)
    gsems = (sga, sgb)

    pltpu.sync_copy(idxh.at[pl.ds(wid * RS, RS)], idxv)

    ILs = (IL1, IL2, IL3, IL4)
    UIs = (UI1, UI2, UI3, UI4)
    plan = []
    for i in range(4):
        plan.append((ILs[i], 1 + i, i))
        plan.append((ILs[i], 5 + i, 4 + i))
        plan.append((UIs[i], 0, 8 + i))
        plan.append((IU, 1 + i, 12 + i))
        plan.append((IU, 5 + i, 16 + i))

    n = len(plan)
    gp = [None, None]
    gp[0] = pltpu.async_copy(plan[0][0].at[idxv.at[plan[0][1]]], bufs[0],
                             gsems[0])
    for j in range(n):
        if j + 1 < n:
            tbl, r, _ = plan[j + 1]
            gp[(j + 1) % 2] = pltpu.async_copy(
                tbl.at[idxv.at[r]], bufs[(j + 1) % 2], gsems[(j + 1) % 2])
        gp[j % 2].wait()
        slot = plan[j][2]
        pltpu.sync_copy(bufs[j % 2], out.at[slot, pl.ds(base, BPW)])


_BB = 512


def _tc_body(f1, f2, f3, f4, r_ref, al_ref, o_ref):
    fmc = [f[...][:, 0:K] * jnp.float32(1.0 / L) for f in (f1, f2, f3, f4)]
    r = r_ref[...]
    a4 = jax.nn.sigmoid(al_ref[0, :])
    neg_inf = jnp.float32(-2.0 ** 32 + 1)
    for i in range(4):
        u = r[8 + i, :, 0:K]
        for sgn in range(2):
            e = r[4 * sgn + i, :, 0:K]
            iu = r[12 + 4 * sgn + i, :, 0:K]
            d = jnp.concatenate(
                [jnp.sum(fmc[t] * e, axis=1, keepdims=True) for t in range(4)],
                axis=1)
            w = d * jnp.float32(0.125)
            w = jnp.where(w == 0.0, neg_inf, w)
            w = w - jnp.max(w, axis=1, keepdims=True)
            p = jnp.exp(w)
            p = p / jnp.sum(p, axis=1, keepdims=True)
            att = jnp.sum(p * d, axis=1)
            mf = jnp.sum(u * iu, axis=1)
            o_ref[2 * i + sgn, :] = a4[i] * att + (1.0 - a4[i]) * mf


def _padt(t):
    return jnp.pad(t, ((0, VPAD - t.shape[0]), (0, K)))


def _stage(rows, stride):
    n = rows.shape[1]
    pad = jnp.zeros((NW, stride - n, 128), jnp.int32)
    return jnp.concatenate([rows, pad], axis=1).reshape(NW * stride, 128)


def kernel(uid, basket_1, basket_2, basket_3, basket_4,
           iid_1, iid_2, iid_3, iid_4,
           neg_iid_1, neg_iid_2, neg_iid_3, neg_iid_4,
           IL_1, IL_2, IL_3, IL_4, LI_1, LI_2, LI_3, LI_4,
           UI_1, UI_2, UI_3, UI_4, IU,
           alpha_mor, alpha_aft, alpha_eve, alpha_deep):
    i32 = jnp.int32

    f = jnp.arange(B * L, dtype=i32)
    srel = (f % (BPW * L)) // L
    sidv = (f // (BPW * L)) // NC
    slot = sidv * BPW + srel
    sah = _stage(slot.reshape(NW, G, 128), GS)
    mhs = [_stage(bk.reshape(-1).astype(i32).reshape(NW, G, 128), GS)
           for bk in (basket_1, basket_2, basket_3, basket_4)]
    zrh = jnp.zeros((BPW, 2 * K), jnp.float32)

    idx9 = jnp.stack([uid, iid_1, iid_2, iid_3, iid_4,
                      neg_iid_1, neg_iid_2, neg_iid_3, neg_iid_4]).astype(i32)
    idxh = _stage(idx9.reshape(9, NW, 128).transpose(1, 0, 2), RS)

    LIp = [_padt(t) for t in (LI_1, LI_2, LI_3, LI_4)]
    ILp = [_padt(t) for t in (IL_1, IL_2, IL_3, IL_4)]
    UIp = [_padt(t) for t in (UI_1, UI_2, UI_3, UI_4)]
    IUp = _padt(IU)

    pooled = [_sc_pool(LIp[i], mhs[i], sah, zrh) for i in range(4)]
    rows = _sc_rows(*ILp, *UIp, IUp, idxh)

    alphas = jnp.stack([alpha_mor, alpha_aft, alpha_eve, alpha_deep])
    alphas = alphas.astype(jnp.float32).reshape(1, 4)

    fspec = pl.BlockSpec((_BB, 2 * K), lambda j: (j, 0))
    out = pl.pallas_call(
        _tc_body,
        grid=(B // _BB,),
        in_specs=[fspec] * 4 + [
            pl.BlockSpec((20, _BB, 2 * K), lambda j: (0, j, 0)),
            pl.BlockSpec((1, 4), lambda j: (0, 0))],
        out_specs=pl.BlockSpec((8, _BB), lambda j: (0, j)),
        out_shape=jax.ShapeDtypeStruct((8, B), jnp.float32),
    )(*pooled, rows, alphas)

    return tuple(out[i] for i in range(8))


# speedup vs baseline: 1.0247x; 1.0178x over previous
import functools

import jax
import jax.numpy as jnp
from jax import lax
from jax.experimental import pallas as pl
from jax.experimental.pallas import tpu as pltpu
from jax.experimental.pallas import tpu_sc as plsc

B = 4096
L = 20
K = 64
NC = 2
NS = 16
NW = NC * NS
BPW = B // NW
G = (BPW * L) // 128

_mesh = plsc.VectorSubcoreMesh(core_axis_name="c", subcore_axis_name="s")
_sc_params = pltpu.CompilerParams(use_tc_tiling_on_sc=False)


@functools.partial(
    pl.kernel,
    mesh=_mesh,
    compiler_params=_sc_params,
    out_type=jax.ShapeDtypeStruct((B, K), jnp.float32),
    scratch_types=[
        pltpu.VMEM((G, 128), jnp.int32),
        pltpu.VMEM((G, 128), jnp.int32),
        pltpu.VMEM((128, K), jnp.float32),
        pltpu.VMEM((128, K), jnp.float32),
        pltpu.VMEM_SHARED((NS * BPW, K), jnp.float32),
        pltpu.SemaphoreType.DMA,
        pltpu.SemaphoreType.DMA,
    ],
)
def _sc_pool(LI, bk, path, zrh, out, bkv, patv, gbufa, gbufb, shacc, sga, sgb):
    wid = lax.axis_index("s") * NC + lax.axis_index("c")
    sid = lax.axis_index("s")
    base = wid * BPW
    bufs = (gbufa, gbufb)
    sems = (sga, sgb)

    pltpu.sync_copy(path.at[sid], patv)
    pltpu.sync_copy(bk.at[wid], bkv)
    pltpu.sync_copy(zrh, shacc.at[pl.ds(sid * BPW, BPW)])

    pend = [None, None]
    pend[0] = pltpu.async_copy(LI.at[bkv.at[0]], bufs[0], sems[0])
    for g in range(G):
        if g + 1 < G:
            pend[(g + 1) % 2] = pltpu.async_copy(
                LI.at[bkv.at[g + 1]], bufs[(g + 1) % 2], sems[(g + 1) % 2])
        pend[g % 2].wait()
        pltpu.sync_copy(bufs[g % 2], shacc.at[patv.at[g]], add=True)

    pltpu.sync_copy(shacc.at[pl.ds(sid * BPW, BPW)], out.at[pl.ds(base, BPW)])


@functools.partial(
    pl.kernel,
    mesh=_mesh,
    compiler_params=_sc_params,
    out_type=jax.ShapeDtypeStruct((20, B, K), jnp.float32),
    scratch_types=[
        pltpu.VMEM((9, 128), jnp.int32),
        pltpu.VMEM((128, K), jnp.float32),
        pltpu.VMEM((128, K), jnp.float32),
        pltpu.SemaphoreType.DMA,
        pltpu.SemaphoreType.DMA,
    ],
)
def _sc_rows(IL1, IL2, IL3, IL4, UI1, UI2, UI3, UI4, IU, idxh,
             out, idxv, gbufa, gbufb, sga, sgb):
    wid = lax.axis_index("s") * NC + lax.axis_index("c")
    base = wid * BPW
    bufs = (gbufa, gbufb)
    gsems = (sga, sgb)

    pltpu.sync_copy(idxh.at[wid], idxv)

    ILs = (IL1, IL2, IL3, IL4)
    UIs = (UI1, UI2, UI3, UI4)
    plan = []
    for i in range(4):
        plan.append((ILs[i], 1 + i, i))
        plan.append((ILs[i], 5 + i, 4 + i))
        plan.append((UIs[i], 0, 8 + i))
        plan.append((IU, 1 + i, 12 + i))
        plan.append((IU, 5 + i, 16 + i))

    n = len(plan)
    gp = [None, None]
    gp[0] = pltpu.async_copy(plan[0][0].at[idxv.at[plan[0][1]]], bufs[0],
                             gsems[0])
    for j in range(n):
        if j + 1 < n:
            tbl, r, _ = plan[j + 1]
            gp[(j + 1) % 2] = pltpu.async_copy(
                tbl.at[idxv.at[r]], bufs[(j + 1) % 2], gsems[(j + 1) % 2])
        gp[j % 2].wait()
        slot = plan[j][2]
        pltpu.sync_copy(bufs[j % 2], out.at[slot, pl.ds(base, BPW)])


_BB = 1024


def _tc_body(f1_ref, f2_ref, f3_ref, f4_ref, r_ref, al_ref, o_ref):
    fmc = [f_ref[...] * jnp.float32(1.0 / L)
           for f_ref in (f1_ref, f2_ref, f3_ref, f4_ref)]
    r = r_ref[...]
    a4 = jax.nn.sigmoid(al_ref[0, :])
    neg_inf = jnp.float32(-2.0 ** 32 + 1)
    for i in range(4):
        u = r[8 + i]
        for sgn in range(2):
            e = r[4 * sgn + i]
            iu = r[12 + 4 * sgn + i]
            d = jnp.concatenate(
                [jnp.sum(fmc[t] * e, axis=1, keepdims=True) for t in range(4)],
                axis=1)
            w = d * jnp.float32(0.125)
            w = jnp.where(w == 0.0, neg_inf, w)
            w = w - jnp.max(w, axis=1, keepdims=True)
            p = jnp.exp(w)
            p = p / jnp.sum(p, axis=1, keepdims=True)
            att = jnp.sum(p * d, axis=1)
            mf = jnp.sum(u * iu, axis=1)
            o_ref[2 * i + sgn, :] = a4[i] * att + (1.0 - a4[i]) * mf


def kernel(uid, basket_1, basket_2, basket_3, basket_4,
           iid_1, iid_2, iid_3, iid_4,
           neg_iid_1, neg_iid_2, neg_iid_3, neg_iid_4,
           IL_1, IL_2, IL_3, IL_4, LI_1, LI_2, LI_3, LI_4,
           UI_1, UI_2, UI_3, UI_4, IU,
           alpha_mor, alpha_aft, alpha_eve, alpha_deep):
    i32 = jnp.int32
    idx_all = jnp.stack([uid, iid_1, iid_2, iid_3, iid_4,
                         neg_iid_1, neg_iid_2, neg_iid_3, neg_iid_4])
    idx_all = idx_all.astype(i32).reshape(9, NW, 128).transpose(1, 0, 2)
    bks = [x.astype(i32).reshape(NW, G, 128)
           for x in (basket_1, basket_2, basket_3, basket_4)]
    pat = (jnp.arange(BPW * L, dtype=i32) // L).reshape(G, 128)
    path = pat[None, :, :] + (jnp.arange(NS, dtype=i32) * BPW)[:, None, None]
    zrh = jnp.zeros((BPW, K), jnp.float32)

    rows = _sc_rows(IL_1, IL_2, IL_3, IL_4, UI_1, UI_2, UI_3, UI_4, IU,
                    idx_all)
    fmcs = [_sc_pool(LIt, bkt, path, zrh)
            for LIt, bkt in zip((LI_1, LI_2, LI_3, LI_4), bks)]

    alphas = jnp.stack([alpha_mor, alpha_aft, alpha_eve, alpha_deep])
    alphas = alphas.astype(jnp.float32).reshape(1, 4)

    fspec = pl.BlockSpec((_BB, K), lambda j: (j, 0))
    out = pl.pallas_call(
        _tc_body,
        grid=(B // _BB,),
        in_specs=[fspec, fspec, fspec, fspec,
                  pl.BlockSpec((20, _BB, K), lambda j: (0, j, 0)),
                  pl.BlockSpec((1, 4), lambda j: (0, 0))],
        out_specs=pl.BlockSpec((8, _BB), lambda j: (0, j)),
        out_shape=jax.ShapeDtypeStruct((8, B), jnp.float32),
    )(*fmcs, rows, alphas)

    return tuple(out[i] for i in range(8))
